# baseline (speedup 1.0000x reference)
import functools

import jax
import jax.numpy as jnp
from jax.experimental import pallas as pl
from jax.experimental.pallas import tpu as pltpu

_SX = 16.0 / 255.0
_INV_SX = 255.0 / 16.0


def _tc_body(x_ref, o_ref):
    x = x_ref[...]
    m = jnp.max(x, axis=-1, keepdims=True)
    q = jnp.clip(jnp.round((x - m) * _INV_SX), -255.0, 0.0)
    e = jnp.exp(q * _SX)
    s = jnp.sum(e, axis=-1, keepdims=True)
    y = e / s
    qy = jnp.clip(jnp.round(y * 255.0 - 128.0), -128.0, 127.0)
    o_ref[...] = (qy + 128.0) * (1.0 / 255.0)


def kernel(inputs):
    b, h, w, w2 = inputs.shape
    rows = b * h * w
    x = inputs.reshape(rows // 64, 64, w2)
    blk = 32
    out = pl.pallas_call(
        _tc_body,
        grid=(x.shape[0] // blk,),
        in_specs=[pl.BlockSpec((blk, 64, w2), lambda i: (i, 0, 0))],
        out_specs=pl.BlockSpec((blk, 64, w2), lambda i: (i, 0, 0)),
        out_shape=jax.ShapeDtypeStruct(x.shape, x.dtype),
    )(x)
    return out.reshape(b, h, w, w2)
